# Initial kernel scaffold; baseline (speedup 1.0000x reference)
import jax
import jax.numpy as jnp
from jax.experimental import pallas as pl


def kernel(user_tensor, item_tensor, edge_index, Wu, bu, Wi, bi, W1, b1, W2, b2):
    raise NotImplementedError("write your pallas kernel here")


# speedup vs baseline: 23.6311x; 23.6311x over previous
import functools

import jax
import jax.numpy as jnp
from jax import lax
from jax.experimental import pallas as pl
from jax.experimental.pallas import tpu as pltpu
from jax.experimental.pallas import tpu_sc as plsc

N_USERS = 5000
N_ITEMS = 5000
N = N_USERS + N_ITEMS
NPAD = 10240
E = 320000
BLK = 80
EROWS = E // BLK
NC = 2
NS = 16
NW = NC * NS
NBLK = EROWS // NW
RPT = NPAD // NS


def _sc_mesh():
  return plsc.VectorSubcoreMesh(core_axis_name="c", subcore_axis_name="s")


@functools.partial(
    pl.kernel,
    out_type=jax.ShapeDtypeStruct((NC, NPAD, 1), jnp.float32),
    mesh=_sc_mesh(),
    scratch_types=[
        pltpu.VMEM((NBLK, BLK), jnp.int32),
        pltpu.VMEM((BLK, 1), jnp.float32),
        pltpu.VMEM_SHARED((NPAD, 1), jnp.float32),
    ],
)
def _sc_deg(dst_hbm, ones_hbm, zero_hbm, out_hbm, didx, ones_v, acc):
  c = lax.axis_index("c")
  s = lax.axis_index("s")
  w = c * NS + s
  pltpu.sync_copy(dst_hbm.at[w], didx)
  pltpu.sync_copy(ones_hbm, ones_v)
  pltpu.sync_copy(zero_hbm.at[pl.ds(s * RPT, RPT)], acc.at[pl.ds(s * RPT, RPT)])
  plsc.subcore_barrier()

  def body(j, carry):
    pltpu.sync_copy(ones_v, acc.at[didx.at[j]], add=True)
    return carry

  lax.fori_loop(0, NBLK, body, 0)
  plsc.subcore_barrier()
  pltpu.sync_copy(acc.at[pl.ds(s * RPT, RPT)],
                  out_hbm.at[c, pl.ds(s * RPT, RPT)])


def _make_prop(D):
  @functools.partial(
      pl.kernel,
      out_type=jax.ShapeDtypeStruct((NC, NPAD, D), jnp.float32),
      mesh=_sc_mesh(),
      scratch_types=[
          pltpu.VMEM((NBLK, BLK), jnp.int32),
          pltpu.VMEM((NBLK, BLK), jnp.int32),
          pltpu.VMEM((BLK, D), jnp.float32),
          pltpu.SemaphoreType.DMA,
          pltpu.VMEM_SHARED((NPAD, D), jnp.float32),
      ],
      compiler_params=pltpu.CompilerParams(use_tc_tiling_on_sc=False),
  )
  def prop(y_hbm, src_hbm, dst_hbm, zero_hbm, out_hbm, sidx, didx, rows, sem,
           acc):
    c = lax.axis_index("c")
    s = lax.axis_index("s")
    w = c * NS + s
    pltpu.sync_copy(src_hbm.at[w], sidx)
    pltpu.sync_copy(dst_hbm.at[w], didx)
    pltpu.sync_copy(zero_hbm.at[pl.ds(s * RPT, RPT)],
                    acc.at[pl.ds(s * RPT, RPT)])
    plsc.subcore_barrier()

    def body(j, carry):
      pltpu.async_copy(y_hbm.at[sidx.at[j]], rows, sem).wait()
      pltpu.sync_copy(rows, acc.at[didx.at[j]], add=True)
      return carry

    lax.fori_loop(0, NBLK, body, 0)
    plsc.subcore_barrier()
    pltpu.sync_copy(acc.at[pl.ds(s * RPT, RPT)],
                    out_hbm.at[c, pl.ds(s * RPT, RPT)])

  return prop


_prop64 = _make_prop(64)
_prop32 = _make_prop(32)


R = 1000
GRID = N // R


def _tc1_body(user_ref, item_ref, wu_ref, bu_ref, wi_ref, bi_ref, w1_ref,
              degp_ref, y1_ref, dis_ref):
  i = pl.program_id(0)
  is_user = i < (N_USERS // R)
  x = jnp.where(is_user, user_ref[...], item_ref[...])
  w = jnp.where(is_user, wu_ref[...], wi_ref[...])
  b = jnp.where(is_user, bu_ref[...], bi_ref[...])
  emb = jnp.dot(x, w, preferred_element_type=jnp.float32) + b
  deg = degp_ref[0] + degp_ref[1] + 1.0
  dis = lax.rsqrt(deg)
  y1 = jnp.dot(emb, w1_ref[...], preferred_element_type=jnp.float32) * dis
  y1_ref[...] = y1
  dis_ref[...] = dis


def _tc1(user, item, wu, bu, wi, bi, w1, degp):
  full = lambda shape: pl.BlockSpec(shape, lambda i: (0,) * len(shape))
  return pl.pallas_call(
      _tc1_body,
      grid=(GRID,),
      in_specs=[
          pl.BlockSpec((R, 128), lambda i: (i % (N_USERS // R), 0)),
          pl.BlockSpec((R, 128), lambda i: (i % (N_ITEMS // R), 0)),
          full((128, 64)),
          full((1, 64)),
          full((128, 64)),
          full((1, 64)),
          full((64, 64)),
          pl.BlockSpec((NC, R, 1), lambda i: (0, i, 0)),
      ],
      out_specs=[
          pl.BlockSpec((R, 64), lambda i: (i, 0)),
          pl.BlockSpec((R, 1), lambda i: (i, 0)),
      ],
      out_shape=[
          jax.ShapeDtypeStruct((N, 64), jnp.float32),
          jax.ShapeDtypeStruct((N, 1), jnp.float32),
      ],
  )(user, item, wu, bu, wi, bi, w1, degp)


def _tc2_body(y1_ref, p_ref, dis_ref, b1_ref, w2_ref, y2_ref):
  dis = dis_ref[...]
  h = dis * (p_ref[0] + p_ref[1] + y1_ref[...]) + b1_ref[...]
  h = jnp.maximum(h, 0.0)
  y2_ref[...] = jnp.dot(h, w2_ref[...], preferred_element_type=jnp.float32) * dis


def _tc2(y1, p1, dis, b1, w2):
  full = lambda shape: pl.BlockSpec(shape, lambda i: (0,) * len(shape))
  return pl.pallas_call(
      _tc2_body,
      grid=(GRID,),
      in_specs=[
          pl.BlockSpec((R, 64), lambda i: (i, 0)),
          pl.BlockSpec((NC, R, 64), lambda i: (0, i, 0)),
          pl.BlockSpec((R, 1), lambda i: (i, 0)),
          full((1, 64)),
          full((64, 32)),
      ],
      out_specs=pl.BlockSpec((R, 32), lambda i: (i, 0)),
      out_shape=jax.ShapeDtypeStruct((N, 32), jnp.float32),
  )(y1, p1, dis, b1, w2)


def _tc3_body(y2_ref, p_ref, dis_ref, b2_ref, out_ref):
  out_ref[...] = (dis_ref[...] * (p_ref[0] + p_ref[1] + y2_ref[...])
                  + b2_ref[...])


def _tc3(y2, p2, dis, b2):
  full = lambda shape: pl.BlockSpec(shape, lambda i: (0,) * len(shape))
  return pl.pallas_call(
      _tc3_body,
      grid=(GRID,),
      in_specs=[
          pl.BlockSpec((R, 32), lambda i: (i, 0)),
          pl.BlockSpec((NC, R, 32), lambda i: (0, i, 0)),
          pl.BlockSpec((R, 1), lambda i: (i, 0)),
          full((1, 32)),
      ],
      out_specs=pl.BlockSpec((R, 32), lambda i: (i, 0)),
      out_shape=jax.ShapeDtypeStruct((N, 32), jnp.float32),
  )(y2, p2, dis, b2)


def kernel(user_tensor, item_tensor, edge_index, Wu, bu, Wi, bi, W1, b1, W2,
           b2):
  src = edge_index[0].reshape(NW, NBLK, BLK)
  dst = edge_index[1].reshape(NW, NBLK, BLK)
  ones_col = jnp.ones((BLK, 1), jnp.float32)
  z1 = jnp.zeros((NPAD, 1), jnp.float32)
  z64 = jnp.zeros((NPAD, 64), jnp.float32)
  z32 = jnp.zeros((NPAD, 32), jnp.float32)

  pad = lambda a: jnp.pad(a, ((0, NPAD - N), (0, 0)))

  degp = _sc_deg(dst, ones_col, z1)
  y1, dis = _tc1(user_tensor, item_tensor, Wu, bu.reshape(1, -1), Wi,
                 bi.reshape(1, -1), W1, degp[:, :N])
  p1 = _prop64(pad(y1), src, dst, z64)
  y2 = _tc2(y1, p1[:, :N], dis, b1.reshape(1, -1), W2)
  p2 = _prop32(pad(y2), src, dst, z32)
  return _tc3(y2, p2[:, :N], dis, b2.reshape(1, -1))


# speedup vs baseline: 24.0015x; 1.0157x over previous
import functools

import jax
import jax.numpy as jnp
from jax import lax
from jax.experimental import pallas as pl
from jax.experimental.pallas import tpu as pltpu
from jax.experimental.pallas import tpu_sc as plsc

N_USERS = 5000
N_ITEMS = 5000
N = N_USERS + N_ITEMS
NPAD = 10240
E = 320000
BLK = 80
EROWS = E // BLK
NC = 2
NS = 16
NW = NC * NS
NBLK = EROWS // NW
RPT = NPAD // NS
NB = 2
NT = (NBLK - 1) // (2 * NB)


def _sc_mesh():
  return plsc.VectorSubcoreMesh(core_axis_name="c", subcore_axis_name="s")


_SC_PARAMS = pltpu.CompilerParams(use_tc_tiling_on_sc=False)


DEGW = 8


@functools.partial(
    pl.kernel,
    out_type=jax.ShapeDtypeStruct((NC, NPAD, DEGW), jnp.float32),
    mesh=_sc_mesh(),
    scratch_types=[
        pltpu.VMEM((NBLK, BLK), jnp.int32),
        pltpu.VMEM((BLK, DEGW), jnp.float32),
        pltpu.VMEM_SHARED((NPAD, DEGW), jnp.float32),
    ],
    compiler_params=pltpu.CompilerParams(use_tc_tiling_on_sc=False),
)
def _sc_deg(dst_hbm, ones_hbm, zero_hbm, out_hbm, didx, ones_v, acc):
  c = lax.axis_index("c")
  s = lax.axis_index("s")
  w = c * NS + s
  pltpu.sync_copy(dst_hbm.at[w], didx)
  pltpu.sync_copy(ones_hbm, ones_v)
  pltpu.sync_copy(zero_hbm.at[pl.ds(s * RPT, RPT)], acc.at[pl.ds(s * RPT, RPT)])
  plsc.subcore_barrier()

  def body(j, carry):
    pltpu.sync_copy(ones_v, acc.at[didx.at[j]], add=True)
    return carry

  lax.fori_loop(0, NBLK, body, 0)
  plsc.subcore_barrier()
  pltpu.sync_copy(acc.at[pl.ds(s * RPT, RPT)],
                  out_hbm.at[c, pl.ds(s * RPT, RPT)])


def _make_prop(D):
  @functools.partial(
      pl.kernel,
      out_type=jax.ShapeDtypeStruct((NC, NPAD, D), jnp.float32),
      mesh=_sc_mesh(),
      scratch_types=[
          pltpu.VMEM((NBLK, BLK), jnp.int32),
          pltpu.VMEM((NBLK, BLK), jnp.int32),
          pltpu.VMEM((BLK, D), jnp.float32),
          pltpu.SemaphoreType.DMA,
          pltpu.VMEM_SHARED((NPAD, D), jnp.float32),
      ],
      compiler_params=_SC_PARAMS,
  )
  def prop(y_hbm, src_hbm, dst_hbm, zero_hbm, out_hbm, sidx, didx, rows,
           gsem0, acc):
    c = lax.axis_index("c")
    s = lax.axis_index("s")
    w = c * NS + s
    pltpu.sync_copy(src_hbm.at[w], sidx)
    pltpu.sync_copy(dst_hbm.at[w], didx)
    pltpu.sync_copy(zero_hbm.at[pl.ds(s * RPT, RPT)],
                    acc.at[pl.ds(s * RPT, RPT)])
    plsc.subcore_barrier()

    def body(j, carry):
      pltpu.async_copy(y_hbm.at[sidx.at[j]], rows, gsem0).wait()
      pltpu.sync_copy(rows, acc.at[didx.at[j]], add=True)
      return carry

    lax.fori_loop(0, NBLK, body, 0)
    plsc.subcore_barrier()
    pltpu.sync_copy(acc.at[pl.ds(s * RPT, RPT)],
                    out_hbm.at[c, pl.ds(s * RPT, RPT)])

  return prop


_prop64 = _make_prop(64)
_prop32 = _make_prop(32)


R = 1000
GRID = N // R


def _tc1_body(user_ref, item_ref, wu_ref, bu_ref, wi_ref, bi_ref, w1_ref,
              degp_ref, y1_ref, dis_ref):
  i = pl.program_id(0)
  is_user = i < (N_USERS // R)
  x = jnp.where(is_user, user_ref[...], item_ref[...])
  w = jnp.where(is_user, wu_ref[...], wi_ref[...])
  b = jnp.where(is_user, bu_ref[...], bi_ref[...])
  emb = jnp.dot(x, w, preferred_element_type=jnp.float32) + b
  deg = degp_ref[0][:, 0:1] + degp_ref[1][:, 0:1] + 1.0
  dis = lax.rsqrt(deg)
  y1 = jnp.dot(emb, w1_ref[...], preferred_element_type=jnp.float32) * dis
  y1_ref[...] = y1
  dis_ref[...] = dis


def _tc1(user, item, wu, bu, wi, bi, w1, degp):
  full = lambda shape: pl.BlockSpec(shape, lambda i: (0,) * len(shape))
  return pl.pallas_call(
      _tc1_body,
      grid=(GRID,),
      in_specs=[
          pl.BlockSpec((R, 128), lambda i: (i % (N_USERS // R), 0)),
          pl.BlockSpec((R, 128), lambda i: (i % (N_ITEMS // R), 0)),
          full((128, 64)),
          full((1, 64)),
          full((128, 64)),
          full((1, 64)),
          full((64, 64)),
          pl.BlockSpec((NC, R, DEGW), lambda i: (0, i, 0)),
      ],
      out_specs=[
          pl.BlockSpec((R, 64), lambda i: (i, 0)),
          pl.BlockSpec((R, 1), lambda i: (i, 0)),
      ],
      out_shape=[
          jax.ShapeDtypeStruct((N, 64), jnp.float32),
          jax.ShapeDtypeStruct((N, 1), jnp.float32),
      ],
  )(user, item, wu, bu, wi, bi, w1, degp)


def _tc2_body(y1_ref, p_ref, dis_ref, b1_ref, w2_ref, y2_ref):
  dis = dis_ref[...]
  h = dis * (p_ref[0] + p_ref[1] + y1_ref[...]) + b1_ref[...]
  h = jnp.maximum(h, 0.0)
  y2_ref[...] = jnp.dot(h, w2_ref[...], preferred_element_type=jnp.float32) * dis


def _tc2(y1, p1, dis, b1, w2):
  full = lambda shape: pl.BlockSpec(shape, lambda i: (0,) * len(shape))
  return pl.pallas_call(
      _tc2_body,
      grid=(GRID,),
      in_specs=[
          pl.BlockSpec((R, 64), lambda i: (i, 0)),
          pl.BlockSpec((NC, R, 64), lambda i: (0, i, 0)),
          pl.BlockSpec((R, 1), lambda i: (i, 0)),
          full((1, 64)),
          full((64, 32)),
      ],
      out_specs=pl.BlockSpec((R, 32), lambda i: (i, 0)),
      out_shape=jax.ShapeDtypeStruct((N, 32), jnp.float32),
  )(y1, p1, dis, b1, w2)


def _tc3_body(y2_ref, p_ref, dis_ref, b2_ref, out_ref):
  out_ref[...] = (dis_ref[...] * (p_ref[0] + p_ref[1] + y2_ref[...])
                  + b2_ref[...])


def _tc3(y2, p2, dis, b2):
  full = lambda shape: pl.BlockSpec(shape, lambda i: (0,) * len(shape))
  return pl.pallas_call(
      _tc3_body,
      grid=(GRID,),
      in_specs=[
          pl.BlockSpec((R, 32), lambda i: (i, 0)),
          pl.BlockSpec((NC, R, 32), lambda i: (0, i, 0)),
          pl.BlockSpec((R, 1), lambda i: (i, 0)),
          full((1, 32)),
      ],
      out_specs=pl.BlockSpec((R, 32), lambda i: (i, 0)),
      out_shape=jax.ShapeDtypeStruct((N, 32), jnp.float32),
  )(y2, p2, dis, b2)


def kernel(user_tensor, item_tensor, edge_index, Wu, bu, Wi, bi, W1, b1, W2,
           b2):
  src = edge_index[0].reshape(NW, NBLK, BLK)
  dst = edge_index[1].reshape(NW, NBLK, BLK)
  ones_col = jnp.ones((BLK, DEGW), jnp.float32)
  z1 = jnp.zeros((NPAD, DEGW), jnp.float32)
  z64 = jnp.zeros((NPAD, 64), jnp.float32)
  z32 = jnp.zeros((NPAD, 32), jnp.float32)

  pad = lambda a: jnp.pad(a, ((0, NPAD - N), (0, 0)))

  degp = _sc_deg(dst, ones_col, z1)
  y1, dis = _tc1(user_tensor, item_tensor, Wu, bu.reshape(1, -1), Wi,
                 bi.reshape(1, -1), W1, degp[:, :N])
  p1 = _prop64(pad(y1), src, dst, z64)
  y2 = _tc2(y1, p1[:, :N], dis, b1.reshape(1, -1), W2)
  p2 = _prop32(pad(y2), src, dst, z32)
  return _tc3(y2, p2[:, :N], dis, b2.reshape(1, -1))


# speedup vs baseline: 35.4948x; 1.4789x over previous
import functools

import jax
import jax.numpy as jnp
from jax import lax
from jax.experimental import pallas as pl
from jax.experimental.pallas import tpu as pltpu
from jax.experimental.pallas import tpu_sc as plsc

N_USERS = 5000
N_ITEMS = 5000
N = N_USERS + N_ITEMS
NPAD = 10240
E = 320000
BLK = 80
EROWS = E // BLK
NC = 2
NS = 16
NW = NC * NS
NBLK = EROWS // NW
RPT = NPAD // NS
NB = 2
NT = (NBLK - 1) // (2 * NB)


def _sc_mesh():
  return plsc.VectorSubcoreMesh(core_axis_name="c", subcore_axis_name="s")


_SC_PARAMS = pltpu.CompilerParams(use_tc_tiling_on_sc=False)


DEGW = 8


@functools.partial(
    pl.kernel,
    out_type=jax.ShapeDtypeStruct((NC, NPAD, DEGW), jnp.float32),
    mesh=_sc_mesh(),
    scratch_types=[
        pltpu.VMEM((NBLK, BLK), jnp.int32),
        pltpu.VMEM((BLK, DEGW), jnp.float32),
        pltpu.SemaphoreType.DMA,
        pltpu.VMEM_SHARED((NPAD, DEGW), jnp.float32),
    ],
    compiler_params=pltpu.CompilerParams(use_tc_tiling_on_sc=False),
)
def _sc_deg(dst_hbm, ones_hbm, zero_hbm, out_hbm, didx, ones_v, sem, acc):
  c = lax.axis_index("c")
  s = lax.axis_index("s")
  w = c * NS + s
  pltpu.sync_copy(dst_hbm.at[w], didx)
  pltpu.sync_copy(ones_hbm, ones_v)
  pltpu.sync_copy(zero_hbm.at[pl.ds(s * RPT, RPT)], acc.at[pl.ds(s * RPT, RPT)])
  plsc.subcore_barrier()

  K = 5

  def body(t, carry):
    for b in range(K):
      pltpu.async_copy(ones_v, acc.at[didx.at[t * K + b]], sem, add=True)
    for b in range(K):
      pltpu.make_async_copy(ones_v, acc.at[didx.at[0]], sem).wait()
    return carry

  lax.fori_loop(0, NBLK // K, body, 0)
  plsc.subcore_barrier()
  pltpu.sync_copy(acc.at[pl.ds(s * RPT, RPT)],
                  out_hbm.at[c, pl.ds(s * RPT, RPT)])


def _make_prop(D):
  @functools.partial(
      pl.kernel,
      out_type=jax.ShapeDtypeStruct((NC, NPAD, D), jnp.float32),
      mesh=_sc_mesh(),
      scratch_types=[
          pltpu.VMEM((NBLK, BLK), jnp.int32),
          pltpu.VMEM((NBLK, BLK), jnp.int32),
          pltpu.VMEM((2, NB, BLK, D), jnp.float32),
          pltpu.SemaphoreType.DMA,
          pltpu.SemaphoreType.DMA,
          pltpu.SemaphoreType.DMA,
          pltpu.SemaphoreType.DMA,
          pltpu.VMEM_SHARED((NPAD, D), jnp.float32),
      ],
      compiler_params=_SC_PARAMS,
  )
  def prop(y_hbm, src_hbm, dst_hbm, zero_hbm, out_hbm, sidx, didx, rows,
           gsem0, gsem1, ssem0, ssem1, acc):
    c = lax.axis_index("c")
    s = lax.axis_index("s")
    w = c * NS + s
    pltpu.sync_copy(src_hbm.at[w], sidx)
    pltpu.sync_copy(dst_hbm.at[w], didx)
    pltpu.sync_copy(zero_hbm.at[pl.ds(s * RPT, RPT)],
                    acc.at[pl.ds(s * RPT, RPT)])
    plsc.subcore_barrier()

    gsem = (gsem0, gsem1)
    ssem = (ssem0, ssem1)

    def issue_g(g, p):
      for b in range(NB):
        pltpu.async_copy(y_hbm.at[sidx.at[g * NB + b]], rows.at[p, b],
                         gsem[p])

    def wait_g(p):
      for b in range(NB):
        pltpu.make_async_copy(y_hbm.at[sidx.at[0]], rows.at[p, b],
                              gsem[p]).wait()

    def issue_s(g, p):
      for b in range(NB):
        pltpu.async_copy(rows.at[p, b], acc.at[didx.at[g * NB + b]], ssem[p],
                         add=True)

    def wait_s(p):
      for b in range(NB):
        pltpu.make_async_copy(rows.at[p, b], acc.at[didx.at[0]],
                              ssem[p]).wait()

    issue_g(0, 0)

    def body(t, carry):
      issue_g(2 * t + 1, 1)
      wait_g(0)
      issue_s(2 * t, 0)
      wait_g(1)
      issue_s(2 * t + 1, 1)
      wait_s(0)

      @pl.when(t + 1 < NT)
      def _():
        issue_g(2 * t + 2, 0)

      wait_s(1)
      return carry

    lax.fori_loop(0, NT, body, 0)
    pltpu.async_copy(y_hbm.at[sidx.at[NBLK - 1]], rows.at[0, 0], gsem0)
    pltpu.make_async_copy(y_hbm.at[sidx.at[0]], rows.at[0, 0], gsem0).wait()
    pltpu.async_copy(rows.at[0, 0], acc.at[didx.at[NBLK - 1]], ssem0,
                     add=True)
    pltpu.make_async_copy(rows.at[0, 0], acc.at[didx.at[0]], ssem0).wait()
    plsc.subcore_barrier()
    pltpu.sync_copy(acc.at[pl.ds(s * RPT, RPT)],
                    out_hbm.at[c, pl.ds(s * RPT, RPT)])

  return prop


_prop64 = _make_prop(64)
_prop32 = _make_prop(32)


R = 1000
GRID = N // R


def _tc1_body(user_ref, item_ref, wu_ref, bu_ref, wi_ref, bi_ref, w1_ref,
              degp_ref, y1_ref, dis_ref):
  i = pl.program_id(0)
  is_user = i < (N_USERS // R)
  x = jnp.where(is_user, user_ref[...], item_ref[...])
  w = jnp.where(is_user, wu_ref[...], wi_ref[...])
  b = jnp.where(is_user, bu_ref[...], bi_ref[...])
  emb = jnp.dot(x, w, preferred_element_type=jnp.float32) + b
  deg = degp_ref[0][:, 0:1] + degp_ref[1][:, 0:1] + 1.0
  dis = lax.rsqrt(deg)
  y1 = jnp.dot(emb, w1_ref[...], preferred_element_type=jnp.float32) * dis
  y1_ref[...] = y1
  dis_ref[...] = dis


def _tc1(user, item, wu, bu, wi, bi, w1, degp):
  full = lambda shape: pl.BlockSpec(shape, lambda i: (0,) * len(shape))
  return pl.pallas_call(
      _tc1_body,
      grid=(GRID,),
      in_specs=[
          pl.BlockSpec((R, 128), lambda i: (i % (N_USERS // R), 0)),
          pl.BlockSpec((R, 128), lambda i: (i % (N_ITEMS // R), 0)),
          full((128, 64)),
          full((1, 64)),
          full((128, 64)),
          full((1, 64)),
          full((64, 64)),
          pl.BlockSpec((NC, R, DEGW), lambda i: (0, i, 0)),
      ],
      out_specs=[
          pl.BlockSpec((R, 64), lambda i: (i, 0)),
          pl.BlockSpec((R, 1), lambda i: (i, 0)),
      ],
      out_shape=[
          jax.ShapeDtypeStruct((N, 64), jnp.float32),
          jax.ShapeDtypeStruct((N, 1), jnp.float32),
      ],
  )(user, item, wu, bu, wi, bi, w1, degp)


def _tc2_body(y1_ref, p_ref, dis_ref, b1_ref, w2_ref, y2_ref):
  dis = dis_ref[...]
  h = dis * (p_ref[0] + p_ref[1] + y1_ref[...]) + b1_ref[...]
  h = jnp.maximum(h, 0.0)
  y2_ref[...] = jnp.dot(h, w2_ref[...], preferred_element_type=jnp.float32) * dis


def _tc2(y1, p1, dis, b1, w2):
  full = lambda shape: pl.BlockSpec(shape, lambda i: (0,) * len(shape))
  return pl.pallas_call(
      _tc2_body,
      grid=(GRID,),
      in_specs=[
          pl.BlockSpec((R, 64), lambda i: (i, 0)),
          pl.BlockSpec((NC, R, 64), lambda i: (0, i, 0)),
          pl.BlockSpec((R, 1), lambda i: (i, 0)),
          full((1, 64)),
          full((64, 32)),
      ],
      out_specs=pl.BlockSpec((R, 32), lambda i: (i, 0)),
      out_shape=jax.ShapeDtypeStruct((N, 32), jnp.float32),
  )(y1, p1, dis, b1, w2)


def _tc3_body(y2_ref, p_ref, dis_ref, b2_ref, out_ref):
  out_ref[...] = (dis_ref[...] * (p_ref[0] + p_ref[1] + y2_ref[...])
                  + b2_ref[...])


def _tc3(y2, p2, dis, b2):
  full = lambda shape: pl.BlockSpec(shape, lambda i: (0,) * len(shape))
  return pl.pallas_call(
      _tc3_body,
      grid=(GRID,),
      in_specs=[
          pl.BlockSpec((R, 32), lambda i: (i, 0)),
          pl.BlockSpec((NC, R, 32), lambda i: (0, i, 0)),
          pl.BlockSpec((R, 1), lambda i: (i, 0)),
          full((1, 32)),
      ],
      out_specs=pl.BlockSpec((R, 32), lambda i: (i, 0)),
      out_shape=jax.ShapeDtypeStruct((N, 32), jnp.float32),
  )(y2, p2, dis, b2)


def kernel(user_tensor, item_tensor, edge_index, Wu, bu, Wi, bi, W1, b1, W2,
           b2):
  src = edge_index[0].reshape(NW, NBLK, BLK)
  dst = edge_index[1].reshape(NW, NBLK, BLK)
  ones_col = jnp.ones((BLK, DEGW), jnp.float32)
  z1 = jnp.zeros((NPAD, DEGW), jnp.float32)
  z64 = jnp.zeros((NPAD, 64), jnp.float32)
  z32 = jnp.zeros((NPAD, 32), jnp.float32)

  pad = lambda a: jnp.pad(a, ((0, NPAD - N), (0, 0)))

  degp = _sc_deg(dst, ones_col, z1)
  y1, dis = _tc1(user_tensor, item_tensor, Wu, bu.reshape(1, -1), Wi,
                 bi.reshape(1, -1), W1, degp[:, :N])
  p1 = _prop64(pad(y1), src, dst, z64)
  y2 = _tc2(y1, p1[:, :N], dis, b1.reshape(1, -1), W2)
  p2 = _prop32(pad(y2), src, dst, z32)
  return _tc3(y2, p2[:, :N], dis, b2.reshape(1, -1))


# speedup vs baseline: 38.1558x; 1.0750x over previous
import functools

import jax
import jax.numpy as jnp
from jax import lax
from jax.experimental import pallas as pl
from jax.experimental.pallas import tpu as pltpu
from jax.experimental.pallas import tpu_sc as plsc

N_USERS = 5000
N_ITEMS = 5000
N = N_USERS + N_ITEMS
NPAD = 10240
E = 320000
BLK = 80
EROWS = E // BLK
NC = 2
NS = 16
NW = NC * NS
NBLK = EROWS // NW
RPT = NPAD // NS
NB = 2
NT = (NBLK - 1) // (2 * NB)


def _sc_mesh():
  return plsc.VectorSubcoreMesh(core_axis_name="c", subcore_axis_name="s")


_SC_PARAMS = pltpu.CompilerParams(use_tc_tiling_on_sc=False)


DEGW = 8


@functools.partial(
    pl.kernel,
    out_type=jax.ShapeDtypeStruct((NC, NPAD, DEGW), jnp.float32),
    mesh=_sc_mesh(),
    scratch_types=[
        pltpu.VMEM((NBLK, BLK), jnp.int32),
        pltpu.VMEM((BLK, DEGW), jnp.float32),
        pltpu.SemaphoreType.DMA,
        pltpu.VMEM_SHARED((NPAD, DEGW), jnp.float32),
    ],
    compiler_params=pltpu.CompilerParams(use_tc_tiling_on_sc=False),
)
def _sc_deg(dst_hbm, ones_hbm, zero_hbm, out_hbm, didx, ones_v, sem, acc):
  c = lax.axis_index("c")
  s = lax.axis_index("s")
  w = c * NS + s
  pltpu.sync_copy(dst_hbm.at[w], didx)
  pltpu.sync_copy(ones_hbm, ones_v)
  pltpu.sync_copy(zero_hbm.at[pl.ds(s * RPT, RPT)], acc.at[pl.ds(s * RPT, RPT)])
  plsc.subcore_barrier()

  K = 5

  def body(t, carry):
    for b in range(K):
      pltpu.async_copy(ones_v, acc.at[didx.at[t * K + b]], sem, add=True)
    for b in range(K):
      pltpu.make_async_copy(ones_v, acc.at[didx.at[0]], sem).wait()
    return carry

  lax.fori_loop(0, NBLK // K, body, 0)
  plsc.subcore_barrier()
  pltpu.sync_copy(acc.at[pl.ds(s * RPT, RPT)],
                  out_hbm.at[c, pl.ds(s * RPT, RPT)])


def _make_prop(D):
  @functools.partial(
      pl.kernel,
      out_type=jax.ShapeDtypeStruct((NC, NPAD, D), jnp.float32),
      mesh=_sc_mesh(),
      scratch_types=[
          pltpu.VMEM((NBLK, BLK), jnp.int32),
          pltpu.VMEM((NBLK, BLK), jnp.int32),
          pltpu.VMEM((2, NB, BLK, D), jnp.float32),
          pltpu.SemaphoreType.DMA,
          pltpu.SemaphoreType.DMA,
          pltpu.SemaphoreType.DMA,
          pltpu.SemaphoreType.DMA,
          pltpu.VMEM_SHARED((NPAD, D), jnp.float32),
      ],
      compiler_params=_SC_PARAMS,
  )
  def prop(y_hbm, src_hbm, dst_hbm, zero_hbm, out_hbm, sidx, didx, rows,
           gsem0, gsem1, ssem0, ssem1, acc):
    c = lax.axis_index("c")
    s = lax.axis_index("s")
    w = c * NS + s
    pltpu.sync_copy(src_hbm.at[w], sidx)
    pltpu.sync_copy(dst_hbm.at[w], didx)
    pltpu.sync_copy(zero_hbm.at[pl.ds(s * RPT, RPT)],
                    acc.at[pl.ds(s * RPT, RPT)])
    plsc.subcore_barrier()

    gsem = (gsem0, gsem1)
    ssem = (ssem0, ssem1)

    def issue_g(g, p):
      for b in range(NB):
        pltpu.async_copy(y_hbm.at[sidx.at[g * NB + b]], rows.at[p, b],
                         gsem[p])

    def wait_g(p):
      for b in range(NB):
        pltpu.make_async_copy(y_hbm.at[sidx.at[0]], rows.at[p, b],
                              gsem[p]).wait()

    def issue_s(g, p):
      for b in range(NB):
        pltpu.async_copy(rows.at[p, b], acc.at[didx.at[g * NB + b]], ssem[p],
                         add=True)

    def wait_s(p):
      for b in range(NB):
        pltpu.make_async_copy(rows.at[p, b], acc.at[didx.at[0]],
                              ssem[p]).wait()

    issue_g(0, 0)

    def body(t, carry):
      issue_g(2 * t + 1, 1)
      wait_g(0)
      issue_s(2 * t, 0)
      wait_g(1)
      issue_s(2 * t + 1, 1)
      wait_s(0)

      @pl.when(t + 1 < NT)
      def _():
        issue_g(2 * t + 2, 0)

      wait_s(1)
      return carry

    lax.fori_loop(0, NT, body, 0)
    pltpu.async_copy(y_hbm.at[sidx.at[NBLK - 1]], rows.at[0, 0], gsem0)
    pltpu.make_async_copy(y_hbm.at[sidx.at[0]], rows.at[0, 0], gsem0).wait()
    pltpu.async_copy(rows.at[0, 0], acc.at[didx.at[NBLK - 1]], ssem0,
                     add=True)
    pltpu.make_async_copy(rows.at[0, 0], acc.at[didx.at[0]], ssem0).wait()
    plsc.subcore_barrier()
    pltpu.sync_copy(acc.at[pl.ds(s * RPT, RPT)],
                    out_hbm.at[c, pl.ds(s * RPT, RPT)])

  return prop


_prop64 = _make_prop(64)
_prop32 = _make_prop(32)


R = 1000
GRID = N // R


def _tc1_body(user_ref, item_ref, wu_ref, bu_ref, wi_ref, bi_ref, w1_ref,
              degp_ref, y1_ref, dis_ref):
  i = pl.program_id(0)
  is_user = i < (N_USERS // R)
  x = jnp.where(is_user, user_ref[...], item_ref[...])
  w = jnp.where(is_user, wu_ref[...], wi_ref[...])
  b = jnp.where(is_user, bu_ref[...], bi_ref[...])
  emb = jnp.dot(x, w, preferred_element_type=jnp.float32) + b
  deg = degp_ref[0][:, 0:1] + degp_ref[1][:, 0:1] + 1.0
  dis = lax.rsqrt(deg)
  y1 = jnp.dot(emb, w1_ref[...], preferred_element_type=jnp.float32) * dis
  y1_ref[...] = y1
  dis_ref[...] = dis


def _tc1(user, item, wu, bu, wi, bi, w1, degp):
  full = lambda shape: pl.BlockSpec(shape, lambda i: (0,) * len(shape))
  return pl.pallas_call(
      _tc1_body,
      grid=(GRID,),
      in_specs=[
          pl.BlockSpec((R, 128), lambda i: (i % (N_USERS // R), 0)),
          pl.BlockSpec((R, 128), lambda i: (i % (N_ITEMS // R), 0)),
          full((128, 64)),
          full((1, 64)),
          full((128, 64)),
          full((1, 64)),
          full((64, 64)),
          pl.BlockSpec((NC, R, DEGW), lambda i: (0, i, 0)),
      ],
      out_specs=[
          pl.BlockSpec((R, 64), lambda i: (i, 0)),
          pl.BlockSpec((R, 1), lambda i: (i, 0)),
      ],
      out_shape=[
          jax.ShapeDtypeStruct((NPAD, 64), jnp.float32),
          jax.ShapeDtypeStruct((N, 1), jnp.float32),
      ],
  )(user, item, wu, bu, wi, bi, w1, degp)


def _tc2_body(y1_ref, p_ref, dis_ref, b1_ref, w2_ref, y2_ref):
  dis = dis_ref[...]
  h = dis * (p_ref[0] + p_ref[1] + y1_ref[...]) + b1_ref[...]
  h = jnp.maximum(h, 0.0)
  y2_ref[...] = jnp.dot(h, w2_ref[...], preferred_element_type=jnp.float32) * dis


def _tc2(y1, p1, dis, b1, w2):
  full = lambda shape: pl.BlockSpec(shape, lambda i: (0,) * len(shape))
  return pl.pallas_call(
      _tc2_body,
      grid=(GRID,),
      in_specs=[
          pl.BlockSpec((R, 64), lambda i: (i, 0)),
          pl.BlockSpec((NC, R, 64), lambda i: (0, i, 0)),
          pl.BlockSpec((R, 1), lambda i: (i, 0)),
          full((1, 64)),
          full((64, 32)),
      ],
      out_specs=pl.BlockSpec((R, 32), lambda i: (i, 0)),
      out_shape=jax.ShapeDtypeStruct((NPAD, 32), jnp.float32),
  )(y1, p1, dis, b1, w2)


def _tc3_body(y2_ref, p_ref, dis_ref, b2_ref, out_ref):
  out_ref[...] = (dis_ref[...] * (p_ref[0] + p_ref[1] + y2_ref[...])
                  + b2_ref[...])


def _tc3(y2, p2, dis, b2):
  full = lambda shape: pl.BlockSpec(shape, lambda i: (0,) * len(shape))
  return pl.pallas_call(
      _tc3_body,
      grid=(GRID,),
      in_specs=[
          pl.BlockSpec((R, 32), lambda i: (i, 0)),
          pl.BlockSpec((NC, R, 32), lambda i: (0, i, 0)),
          pl.BlockSpec((R, 1), lambda i: (i, 0)),
          full((1, 32)),
      ],
      out_specs=pl.BlockSpec((R, 32), lambda i: (i, 0)),
      out_shape=jax.ShapeDtypeStruct((N, 32), jnp.float32),
  )(y2, p2, dis, b2)


def kernel(user_tensor, item_tensor, edge_index, Wu, bu, Wi, bi, W1, b1, W2,
           b2):
  src = edge_index[0].reshape(NW, NBLK, BLK)
  dst = edge_index[1].reshape(NW, NBLK, BLK)
  ones_col = jnp.ones((BLK, DEGW), jnp.float32)
  z1 = jnp.zeros((NPAD, DEGW), jnp.float32)
  z64 = jnp.zeros((NPAD, 64), jnp.float32)
  z32 = jnp.zeros((NPAD, 32), jnp.float32)

  degp = _sc_deg(dst, ones_col, z1)
  y1, dis = _tc1(user_tensor, item_tensor, Wu, bu.reshape(1, -1), Wi,
                 bi.reshape(1, -1), W1, degp)
  p1 = _prop64(y1, src, dst, z64)
  y2 = _tc2(y1, p1, dis, b1.reshape(1, -1), W2)
  p2 = _prop32(y2, src, dst, z32)
  return _tc3(y2, p2, dis, b2.reshape(1, -1))
